# baseline (speedup 1.0000x reference)
import functools

import jax
import jax.numpy as jnp
from jax.experimental import pallas as pl
from jax.experimental.pallas import tpu as pltpu

_BT_STATS = 4096
_BT_APPLY = 2048


def _xmap(i, seq, bt, bpb):
    b = i // bpb
    k = i % bpb
    lastv = jnp.maximum((seq[b] + bt - 1) // bt - 1, 0)
    return (b, jnp.minimum(k, lastv), 0)


def _stats_kernel(seq_ref, x_ref, g_ref, bet_ref, w_ref, b_ref,
                  w2_ref, aux_ref, acc_ref, *, bt, bpb, nb, nbatch, out_dim):
    i = pl.program_id(0)
    b = i // bpb
    start = (i % bpb) * bt
    seqlen = seq_ref[b]
    valid = seqlen > start

    @pl.when(i == 0)
    def _init():
        acc_ref[...] = jnp.zeros_like(acc_ref)

    @pl.when(valid)
    def _stats():
        pos = start + jax.lax.broadcasted_iota(jnp.int32, (1, bt), 1)
        m = (pos < seqlen).astype(jnp.bfloat16)
        xb = x_ref[0].astype(jnp.bfloat16)
        acc_ref[0:1, :] += jax.lax.dot_general(
            m, xb, (((1,), (0,)), ((), ())),
            preferred_element_type=jnp.float32)
        acc_ref[1:2, :] += jax.lax.dot_general(
            m, xb * xb, (((1,), (0,)), ((), ())),
            preferred_element_type=jnp.float32)

    @pl.when(i == nb - 1)
    def _finalize():
        cnt = jax.lax.fori_loop(
            0, nbatch, lambda k, a: a + seq_ref[k], jnp.int32(0))
        cnt = jnp.maximum(cnt.astype(jnp.float32), 1.0)
        mean = acc_ref[0:1, :] / cnt
        var = acc_ref[1:2, :] / cnt - mean * mean
        scale = jax.lax.rsqrt(var + 1e-5) * g_ref[...][None, :]
        shift = bet_ref[...][None, :] - mean * scale
        w2_ref[...] = (w_ref[...] * scale).astype(jnp.bfloat16)
        brow = b_ref[...][None, :]
        b2 = brow + jax.lax.dot_general(
            shift, w_ref[...], (((1,), (1,)), ((), ())),
            preferred_element_type=jnp.float32)
        bhat = brow * jax.lax.rsqrt(jnp.sum(brow * brow) + 1e-12)
        aux_ref[...] = jax.lax.transpose(
            jnp.concatenate(
                [b2, bhat, jnp.zeros((6, out_dim), jnp.float32)], axis=0),
            (1, 0))


def _apply_kernel(seq_ref, x_ref, w2_ref, aux_ref, out_ref, *, bt, bpb):
    i = pl.program_id(0)
    b = i // bpb
    start = (i % bpb) * bt
    seqlen = seq_ref[b]
    valid = seqlen > start
    full = seqlen >= start + bt

    def _yt():
        return jax.lax.dot_general(
            w2_ref[...], x_ref[0].astype(jnp.bfloat16),
            (((1,), (1,)), ((), ())),
            preferred_element_type=jnp.float32) + aux_ref[:, 0:1]

    @pl.when(full)
    def _apply_full():
        y = _yt()
        out_ref[0] = y * jax.lax.rsqrt(
            jnp.sum(y * y, axis=0, keepdims=True) + 1e-12)

    @pl.when(jnp.logical_and(valid, jnp.logical_not(full)))
    def _apply_partial():
        y = _yt()
        y = y * jax.lax.rsqrt(jnp.sum(y * y, axis=0, keepdims=True) + 1e-12)
        pos = start + jax.lax.broadcasted_iota(jnp.int32, (1, bt), 1)
        out_ref[0] = jnp.where(pos < seqlen, y, aux_ref[:, 1:2])

    @pl.when(jnp.logical_not(valid))
    def _apply_pad():
        out_ref[0] = jnp.broadcast_to(
            aux_ref[:, 1:2], (out_ref.shape[1], bt))


def kernel(payload, seq_lens, gamma, beta, W, b):
    B, T, D = payload.shape
    O = W.shape[0]
    bt = _BT_STATS
    bpb = T // bt
    nb = B * bpb
    bta = _BT_APPLY
    bpba = T // bta
    nba = B * bpba

    seq = seq_lens if seq_lens.dtype == jnp.int32 else seq_lens.astype(jnp.int32)
    xmap = functools.partial(_xmap, bt=bt, bpb=bpb)
    xmapa = functools.partial(_xmap, bt=bta, bpb=bpba)

    w2, aux = pl.pallas_call(
        functools.partial(_stats_kernel, bt=bt, bpb=bpb, nb=nb, nbatch=B,
                          out_dim=O),
        grid_spec=pltpu.PrefetchScalarGridSpec(
            num_scalar_prefetch=1,
            grid=(nb,),
            in_specs=[
                pl.BlockSpec((1, bt, D), xmap),
                pl.BlockSpec((D,), lambda i, seq: (0,)),
                pl.BlockSpec((D,), lambda i, seq: (0,)),
                pl.BlockSpec((O, D), lambda i, seq: (0, 0)),
                pl.BlockSpec((O,), lambda i, seq: (0,)),
            ],
            out_specs=[
                pl.BlockSpec((O, D), lambda i, seq: (0, 0)),
                pl.BlockSpec((O, 8), lambda i, seq: (0, 0)),
            ],
            scratch_shapes=[pltpu.VMEM((8, D), jnp.float32)],
        ),
        out_shape=[
            jax.ShapeDtypeStruct((O, D), jnp.bfloat16),
            jax.ShapeDtypeStruct((O, 8), jnp.float32),
        ],
        compiler_params=pltpu.CompilerParams(
            dimension_semantics=("arbitrary",)),
    )(seq, payload, gamma, beta, W, b)

    y = pl.pallas_call(
        functools.partial(_apply_kernel, bt=bta, bpb=bpba),
        grid_spec=pltpu.PrefetchScalarGridSpec(
            num_scalar_prefetch=1,
            grid=(nba,),
            in_specs=[
                pl.BlockSpec((1, bta, D), xmapa),
                pl.BlockSpec((O, D), lambda i, seq: (0, 0)),
                pl.BlockSpec((O, 8), lambda i, seq: (0, 0)),
            ],
            out_specs=pl.BlockSpec(
                (1, O, bta), lambda i, seq: (i // bpba, 0, i % bpba)),
        ),
        out_shape=jax.ShapeDtypeStruct((B, O, T), jnp.float32),
        compiler_params=pltpu.CompilerParams(
            dimension_semantics=("arbitrary",)),
    )(seq, payload, w2, aux)

    return jnp.swapaxes(y, 1, 2)


# speedup vs baseline: 1.5424x; 1.5424x over previous
import functools

import jax
import jax.numpy as jnp
from jax.experimental import pallas as pl
from jax.experimental.pallas import tpu as pltpu

_SUB = 1024


def _fused_kernel(seq_ref, x_ref, g_ref, bet_ref, w_ref, b_ref, out_ref,
                  xb_ref, acc_ref, w2_ref, aux_ref, *, bt, nb, out_dim):
    i = pl.program_id(0)
    phase1 = i < nb

    @pl.when(i == 0)
    def _init():
        acc_ref[...] = jnp.zeros_like(acc_ref)

    @pl.when(phase1)
    def _phase1():
        seqlen = seq_ref[i]
        xb = x_ref[0].astype(jnp.bfloat16)
        xb_ref[i] = xb

        @pl.when(seqlen > 0)
        def _stats():
            pos = jax.lax.broadcasted_iota(jnp.int32, (1, bt), 1)
            m = (pos < seqlen).astype(jnp.bfloat16)
            acc_ref[0:1, :] += jax.lax.dot_general(
                m, xb, (((1,), (0,)), ((), ())),
                preferred_element_type=jnp.float32)
            acc_ref[1:2, :] += jax.lax.dot_general(
                m, xb * xb, (((1,), (0,)), ((), ())),
                preferred_element_type=jnp.float32)

    @pl.when(i == nb - 1)
    def _finalize():
        cnt = jax.lax.fori_loop(
            0, nb, lambda k, a: a + seq_ref[k], jnp.int32(0))
        cnt = jnp.maximum(cnt.astype(jnp.float32), 1.0)
        mean = acc_ref[0:1, :] / cnt
        var = acc_ref[1:2, :] / cnt - mean * mean
        scale = jax.lax.rsqrt(var + 1e-5) * g_ref[...][None, :]
        shift = bet_ref[...][None, :] - mean * scale
        w2_ref[...] = (w_ref[...] * scale).astype(jnp.bfloat16)
        brow = b_ref[...][None, :]
        b2 = brow + jax.lax.dot_general(
            shift, w_ref[...], (((1,), (1,)), ((), ())),
            preferred_element_type=jnp.float32)
        bhat = brow * jax.lax.rsqrt(jnp.sum(brow * brow) + 1e-12)
        aux_ref[...] = jax.lax.transpose(
            jnp.concatenate(
                [b2, bhat, jnp.zeros((6, out_dim), jnp.float32)], axis=0),
            (1, 0))

    @pl.when(jnp.logical_not(phase1))
    def _phase2():
        j = i - nb
        seqlen = seq_ref[j]
        for s in range(bt // _SUB):
            st = s * _SUB

            @pl.when(seqlen > st)
            def _proj(st=st):
                yt = jax.lax.dot_general(
                    w2_ref[...], xb_ref[j, st:st + _SUB, :],
                    (((1,), (1,)), ((), ())),
                    preferred_element_type=jnp.float32) + aux_ref[:, 0:1]
                yt = yt * jax.lax.rsqrt(
                    jnp.sum(yt * yt, axis=0, keepdims=True) + 1e-12)
                pos = st + jax.lax.broadcasted_iota(jnp.int32, (1, _SUB), 1)
                out_ref[0, :, st:st + _SUB] = jnp.where(
                    pos < seqlen, yt, aux_ref[:, 1:2])

            @pl.when(seqlen <= st)
            def _pad(st=st):
                out_ref[0, :, st:st + _SUB] = jnp.broadcast_to(
                    aux_ref[:, 1:2], (out_dim, _SUB))


def kernel(payload, seq_lens, gamma, beta, W, b):
    B, T, D = payload.shape
    O = W.shape[0]
    bt = T
    nb = B

    seq = seq_lens if seq_lens.dtype == jnp.int32 else seq_lens.astype(jnp.int32)

    y = pl.pallas_call(
        functools.partial(_fused_kernel, bt=bt, nb=nb, out_dim=O),
        grid_spec=pltpu.PrefetchScalarGridSpec(
            num_scalar_prefetch=1,
            grid=(2 * nb,),
            in_specs=[
                pl.BlockSpec((1, bt, D),
                             lambda i, seq: (jnp.minimum(i, nb - 1), 0, 0)),
                pl.BlockSpec((D,), lambda i, seq: (0,)),
                pl.BlockSpec((D,), lambda i, seq: (0,)),
                pl.BlockSpec((O, D), lambda i, seq: (0, 0)),
                pl.BlockSpec((O,), lambda i, seq: (0,)),
            ],
            out_specs=pl.BlockSpec(
                (1, O, bt),
                lambda i, seq: (jnp.maximum(i - nb, 0), 0, 0)),
            scratch_shapes=[
                pltpu.VMEM((nb, bt, D), jnp.bfloat16),
                pltpu.VMEM((8, D), jnp.float32),
                pltpu.VMEM((O, D), jnp.bfloat16),
                pltpu.VMEM((O, 8), jnp.float32),
            ],
        ),
        out_shape=jax.ShapeDtypeStruct((B, O, T), jnp.float32),
        compiler_params=pltpu.CompilerParams(
            dimension_semantics=("arbitrary",)),
    )(seq, payload, gamma, beta, W, b)

    return jnp.swapaxes(y, 1, 2)
